# baseline (speedup 1.0000x reference)
import functools

import jax
import jax.numpy as jnp
from jax import lax
from jax.experimental import pallas as pl
from jax.experimental.pallas import tpu as pltpu
from jax.experimental.pallas import tpu_sc as plsc

_F32 = jnp.float32


def _top3_axis0(dist, S):
    iota = lax.broadcasted_iota(jnp.int32, dist.shape, 0)
    inf = jnp.array(jnp.inf, _F32)
    m1 = jnp.min(dist, axis=0, keepdims=True)
    i1 = jnp.min(jnp.where(dist == m1, iota, S), axis=0, keepdims=True)
    d1 = jnp.where(iota == i1, inf, dist)
    m2 = jnp.min(d1, axis=0, keepdims=True)
    i2 = jnp.min(jnp.where(d1 == m2, iota, S), axis=0, keepdims=True)
    d2 = jnp.where(iota == i2, inf, d1)
    m3 = jnp.min(d2, axis=0, keepdims=True)
    i3 = jnp.min(jnp.where(d2 == m3, iota, S), axis=0, keepdims=True)
    return (m1, m2, m3), (i1, i2, i3)


def _inv_dist_weights(m1, m2, m3):
    d1 = jnp.maximum(m1, 1e-8)
    d2 = jnp.maximum(m2, 1e-8)
    d3 = jnp.maximum(m3, 1e-8)
    w1 = 1.0 / d1
    w2 = 1.0 / d2
    w3 = 1.0 / d3
    s = w1 + w2 + w3
    return w1 / s, w2 / s, w3 / s


def _onehot_t(i123, w123, S, T):
    iota = lax.broadcasted_iota(jnp.int32, (S, T), 0)
    zero = jnp.array(0.0, _F32)
    (i1, i2, i3), (w1, w2, w3) = i123, w123
    return (jnp.where(iota == i1, w1, zero)
            + jnp.where(iota == i2, w2, zero)
            + jnp.where(iota == i3, w3, zero))


def _dist_matrix(src, dstT):
    cross = jnp.dot(src, dstT, preferred_element_type=_F32)
    s2 = jnp.sum(src * src, axis=1, keepdims=True)
    t2 = jnp.sum(dstT * dstT, axis=0, keepdims=True)
    d2 = s2 + t2 - 2.0 * cross
    return jnp.sqrt(jnp.maximum(d2, 0.0))


def _interp_t(ohT, g):
    return lax.dot_general(ohT, g, (((0,), (0,)), ((), ())),
                           preferred_element_type=_F32)


def _a1_body(f1, f2, f3, c2, c3, c1T, c2T,
             w3at, w3ab, b3a, w3b, b3b,
             w2at, w2ab, b2a, w2b, b2b, w1a, g1_out):
    f1v, f2v, f3v = f1[0], f2[0], f3[0]
    c2v, c3v = c2[0], c3[0]
    c1Tv, c2Tv = c1T[0], c2T[0]
    S3, S2, S1 = f3v.shape[0], f2v.shape[0], f1v.shape[0]

    dist = _dist_matrix(c3v, c2Tv)
    ms, is_ = _top3_axis0(dist, S3)
    ws = _inv_dist_weights(*ms)
    ohT = _onehot_t(is_, ws, S3, S2)
    g3 = jnp.dot(f3v, w3ab[...], preferred_element_type=_F32)
    up = _interp_t(ohT, g3)
    skip = jnp.dot(f2v, w3at[...], preferred_element_type=_F32)
    h = jnp.maximum(skip + up + b3a[...], 0.0)
    fused2 = jnp.dot(h, w3b[...], preferred_element_type=_F32) + b3b[...]

    dist = _dist_matrix(c2v, c1Tv)
    ms, is_ = _top3_axis0(dist, S2)
    ws = _inv_dist_weights(*ms)
    ohT = _onehot_t(is_, ws, S2, S1)
    g2 = jnp.dot(fused2, w2ab[...], preferred_element_type=_F32)
    up = _interp_t(ohT, g2)
    skip = jnp.dot(f1v, w2at[...], preferred_element_type=_F32)
    h = jnp.maximum(skip + up + b2a[...], 0.0)
    fused1 = jnp.dot(h, w2b[...], preferred_element_type=_F32) + b2b[...]

    g1_out[0] = jnp.dot(fused1, w1a[...], preferred_element_type=_F32)


def _a2_body(c1, xT, idx_out, w_out, *, S):
    c1v = c1[0]
    xTv = xT[0]
    blk = xTv.shape[1]
    dist = _dist_matrix(c1v, xTv)
    (m1, m2, m3), (i1, i2, i3) = _top3_axis0(dist, S)
    w1, w2, w3 = _inv_dist_weights(m1, m2, m3)
    off = pl.program_id(0) * S
    zi = jnp.zeros((5, blk), jnp.int32)
    zf = jnp.zeros((5, blk), _F32)
    idx_out[0] = jnp.concatenate([i1 + off, i2 + off, i3 + off, zi], axis=0)
    w_out[0] = jnp.concatenate([w1, w2, w3, zf], axis=0)


def _sc_gather_body(g1f, idxf, wf, up, i0_v, i1_v, i2_v, w0_v, w1_v, w2_v,
                    r0_v, r1_v, r2_v, o_v, sem, *, N, C, G, NC):
    b = lax.axis_index("s") * NC + lax.axis_index("c")
    nchunks = N // G
    ccols = C // 16

    def chunk(ci, _):
        base = ci * G
        pltpu.sync_copy(idxf.at[pl.ds((b * 8 + 0) * N + base, G)], i0_v)
        pltpu.sync_copy(idxf.at[pl.ds((b * 8 + 1) * N + base, G)], i1_v)
        pltpu.sync_copy(idxf.at[pl.ds((b * 8 + 2) * N + base, G)], i2_v)
        pltpu.sync_copy(wf.at[pl.ds((b * 8 + 0) * N + base, G)], w0_v)
        pltpu.sync_copy(wf.at[pl.ds((b * 8 + 1) * N + base, G)], w1_v)
        pltpu.sync_copy(wf.at[pl.ds((b * 8 + 2) * N + base, G)], w2_v)
        pltpu.async_copy(g1f.at[i0_v], r0_v, sem).wait()
        pltpu.async_copy(g1f.at[i1_v], r1_v, sem).wait()
        pltpu.async_copy(g1f.at[i2_v], r2_v, sem).wait()

        dn = lax.GatherDimensionNumbers(offset_dims=(), collapsed_slice_dims=(0,),
                                        start_index_map=(0,))
        splat = lambda vec, jidx: lax.gather(
            vec, jidx[:, None], dn, (1,),
            mode=lax.GatherScatterMode.PROMISE_IN_BOUNDS)

        def rowgrp(r, carry):
            g0 = r * 16
            wa0 = w0_v[pl.ds(g0, 16)]
            wa1 = w1_v[pl.ds(g0, 16)]
            wa2 = w2_v[pl.ds(g0, 16)]

            def row(j, carry2):
                g = g0 + j
                jidx = jnp.full((16,), j, jnp.int32)
                wv0 = splat(wa0, jidx)
                wv1 = splat(wa1, jidx)
                wv2 = splat(wa2, jidx)
                for c in range(ccols):
                    sl = pl.ds(c * 16, 16)
                    o_v[g, sl] = (r0_v[g, sl] * wv0 + r1_v[g, sl] * wv1
                                  + r2_v[g, sl] * wv2)
                return carry2

            lax.fori_loop(0, 16, row, None)
            return carry

        lax.fori_loop(0, G // 16, rowgrp, None)
        pltpu.sync_copy(o_v, up.at[pl.ds(b * N + base, G)])
        return _

    lax.fori_loop(0, nchunks, chunk, None, unroll=False)


def _bp_body(up, b1a, w1b, b1b, out):
    h = jnp.maximum(up[...] + b1a[...], 0.0)
    out[...] = jnp.dot(h, w1b[...], preferred_element_type=_F32) + b1b[...]


def kernel(feat1, feat2, feat3, ctr1, ctr2, ctr3, xyz,
           w3a, b3a, w3b, b3b, w2a, b2a, w2b, b2b, w1a, b1a, w1b, b1b):
    B, N, C = feat1.shape[0], xyz.shape[1], feat1.shape[2]
    S1, S2, S3 = feat1.shape[1], feat2.shape[1], feat3.shape[1]

    ctr1T = jnp.swapaxes(ctr1, 1, 2)
    ctr2T = jnp.swapaxes(ctr2, 1, 2)
    xyzT = jnp.swapaxes(xyz, 1, 2)
    w3at, w3ab = w3a[:C], w3a[C:]
    w2at, w2ab = w2a[:C], w2a[C:]
    b3a2 = b3a.reshape(1, C)
    b3b2 = b3b.reshape(1, C)
    b2a2 = b2a.reshape(1, C)
    b2b2 = b2b.reshape(1, C)
    b1a2 = b1a.reshape(1, C)
    b1b2 = b1b.reshape(1, C)

    full = lambda shape: pl.BlockSpec(shape, lambda *_: (0,) * len(shape))
    batch = lambda shape: pl.BlockSpec((1,) + shape,
                                       lambda b, *_: (b,) + (0,) * len(shape))

    g1 = pl.pallas_call(
        _a1_body,
        grid=(B,),
        in_specs=[
            batch((S1, C)), batch((S2, C)), batch((S3, C)),
            batch((S2, 3)), batch((S3, 3)),
            batch((3, S1)), batch((3, S2)),
            full((C, C)), full((C, C)), full((1, C)), full((C, C)), full((1, C)),
            full((C, C)), full((C, C)), full((1, C)), full((C, C)), full((1, C)),
            full((C, C)),
        ],
        out_specs=batch((S1, C)),
        out_shape=jax.ShapeDtypeStruct((B, S1, C), _F32),
    )(feat1, feat2, feat3, ctr2, ctr3, ctr1T, ctr2T,
      w3at, w3ab, b3a2, w3b, b3b2, w2at, w2ab, b2a2, w2b, b2b2, w1a)

    BLK_A = 1024
    idx8, w8 = pl.pallas_call(
        functools.partial(_a2_body, S=S1),
        grid=(B, N // BLK_A),
        in_specs=[
            pl.BlockSpec((1, S1, 3), lambda b, n: (b, 0, 0)),
            pl.BlockSpec((1, 3, BLK_A), lambda b, n: (b, 0, n)),
        ],
        out_specs=[
            pl.BlockSpec((1, 8, BLK_A), lambda b, n: (b, 0, n)),
            pl.BlockSpec((1, 8, BLK_A), lambda b, n: (b, 0, n)),
        ],
        out_shape=[
            jax.ShapeDtypeStruct((B, 8, N), jnp.int32),
            jax.ShapeDtypeStruct((B, 8, N), _F32),
        ],
    )(ctr1, xyzT)

    G = 64
    info = plsc.get_sparse_core_info()
    NC = info.num_cores
    g1f = g1.reshape(B * S1, C)
    idxf = idx8.reshape(B * 8 * N)
    wf = w8.reshape(B * 8 * N)
    mesh = plsc.VectorSubcoreMesh(core_axis_name="c", subcore_axis_name="s")
    up = pl.kernel(
        functools.partial(_sc_gather_body, N=N, C=C, G=G, NC=NC),
        mesh=mesh,
        out_type=jax.ShapeDtypeStruct((B * N, C), _F32),
        scratch_types=[
            pltpu.VMEM((G,), jnp.int32), pltpu.VMEM((G,), jnp.int32),
            pltpu.VMEM((G,), jnp.int32),
            pltpu.VMEM((G,), _F32), pltpu.VMEM((G,), _F32),
            pltpu.VMEM((G,), _F32),
            pltpu.VMEM((G, C), _F32), pltpu.VMEM((G, C), _F32),
            pltpu.VMEM((G, C), _F32), pltpu.VMEM((G, C), _F32),
            pltpu.SemaphoreType.DMA,
        ],
    )(g1f, idxf, wf)

    BLK_B = 1024
    out = pl.pallas_call(
        _bp_body,
        grid=(B * N // BLK_B,),
        in_specs=[
            pl.BlockSpec((BLK_B, C), lambda n: (n, 0)),
            pl.BlockSpec((1, C), lambda n: (0, 0)),
            pl.BlockSpec((C, C), lambda n: (0, 0)),
            pl.BlockSpec((1, C), lambda n: (0, 0)),
        ],
        out_specs=pl.BlockSpec((BLK_B, C), lambda n: (n, 0)),
        out_shape=jax.ShapeDtypeStruct((B * N, C), _F32),
    )(up, b1a2, w1b, b1b2)
    return out.reshape(B, N, C)


# speedup vs baseline: 1.1750x; 1.1750x over previous
import functools

import jax
import jax.numpy as jnp
from jax import lax
from jax.experimental import pallas as pl
from jax.experimental.pallas import tpu as pltpu
from jax.experimental.pallas import tpu_sc as plsc

_F32 = jnp.float32


def _top3_axis0(dist, S):
    iota = lax.broadcasted_iota(jnp.int32, dist.shape, 0)
    inf = jnp.array(jnp.inf, _F32)
    m1 = jnp.min(dist, axis=0, keepdims=True)
    i1 = jnp.min(jnp.where(dist == m1, iota, S), axis=0, keepdims=True)
    d1 = jnp.where(iota == i1, inf, dist)
    m2 = jnp.min(d1, axis=0, keepdims=True)
    i2 = jnp.min(jnp.where(d1 == m2, iota, S), axis=0, keepdims=True)
    d2 = jnp.where(iota == i2, inf, d1)
    m3 = jnp.min(d2, axis=0, keepdims=True)
    i3 = jnp.min(jnp.where(d2 == m3, iota, S), axis=0, keepdims=True)
    return (m1, m2, m3), (i1, i2, i3)


def _inv_dist_weights(m1, m2, m3):
    d1 = jnp.maximum(m1, 1e-8)
    d2 = jnp.maximum(m2, 1e-8)
    d3 = jnp.maximum(m3, 1e-8)
    w1 = 1.0 / d1
    w2 = 1.0 / d2
    w3 = 1.0 / d3
    s = w1 + w2 + w3
    return w1 / s, w2 / s, w3 / s


def _onehot_t(i123, w123, S, T):
    iota = lax.broadcasted_iota(jnp.int32, (S, T), 0)
    zero = jnp.array(0.0, _F32)
    (i1, i2, i3), (w1, w2, w3) = i123, w123
    return (jnp.where(iota == i1, w1, zero)
            + jnp.where(iota == i2, w2, zero)
            + jnp.where(iota == i3, w3, zero))


def _dist_matrix(src, dstT):
    cross = jnp.dot(src, dstT, preferred_element_type=_F32)
    s2 = jnp.sum(src * src, axis=1, keepdims=True)
    t2 = jnp.sum(dstT * dstT, axis=0, keepdims=True)
    d2 = s2 + t2 - 2.0 * cross
    return jnp.sqrt(jnp.maximum(d2, 0.0))


def _interp_t(ohT, g):
    return lax.dot_general(ohT, g, (((0,), (0,)), ((), ())),
                           preferred_element_type=_F32)


def _a1_body(f1, f2, f3, c2, c3, c1T, c2T,
             w3at, w3ab, b3a, w3b, b3b,
             w2at, w2ab, b2a, w2b, b2b, w1a, g1_out):
    f1v, f2v, f3v = f1[0], f2[0], f3[0]
    c2v, c3v = c2[0], c3[0]
    c1Tv, c2Tv = c1T[0], c2T[0]
    S3, S2, S1 = f3v.shape[0], f2v.shape[0], f1v.shape[0]

    dist = _dist_matrix(c3v, c2Tv)
    ms, is_ = _top3_axis0(dist, S3)
    ws = _inv_dist_weights(*ms)
    ohT = _onehot_t(is_, ws, S3, S2)
    g3 = jnp.dot(f3v, w3ab[...], preferred_element_type=_F32)
    up = _interp_t(ohT, g3)
    skip = jnp.dot(f2v, w3at[...], preferred_element_type=_F32)
    h = jnp.maximum(skip + up + b3a[...], 0.0)
    fused2 = jnp.dot(h, w3b[...], preferred_element_type=_F32) + b3b[...]

    dist = _dist_matrix(c2v, c1Tv)
    ms, is_ = _top3_axis0(dist, S2)
    ws = _inv_dist_weights(*ms)
    ohT = _onehot_t(is_, ws, S2, S1)
    g2 = jnp.dot(fused2, w2ab[...], preferred_element_type=_F32)
    up = _interp_t(ohT, g2)
    skip = jnp.dot(f1v, w2at[...], preferred_element_type=_F32)
    h = jnp.maximum(skip + up + b2a[...], 0.0)
    fused1 = jnp.dot(h, w2b[...], preferred_element_type=_F32) + b2b[...]

    g1_out[0] = jnp.dot(fused1, w1a[...], preferred_element_type=_F32)


def _a2_body(c1, xT, idx_out, w_out, *, S):
    c1v = c1[0]
    xTv = xT[0]
    blk = xTv.shape[1]
    dist = _dist_matrix(c1v, xTv)
    (m1, m2, m3), (i1, i2, i3) = _top3_axis0(dist, S)
    w1, w2, w3 = _inv_dist_weights(m1, m2, m3)
    off = pl.program_id(0) * S
    zi = jnp.zeros((5, blk), jnp.int32)
    zf = jnp.zeros((5, blk), _F32)
    idx_out[0] = jnp.concatenate([i1 + off, i2 + off, i3 + off, zi], axis=0)
    w_out[0] = jnp.concatenate([w1, w2, w3, zf], axis=0)


def _sc_gather_body(g1f, idxf, wf, up, i0_v, i1_v, i2_v, w0_v, w1_v, w2_v,
                    r0_v, r1_v, r2_v, sem, *, N, C, G, NC):
    b = lax.axis_index("s") * NC + lax.axis_index("c")
    nchunks = N // G
    ccols = C // 16

    pltpu.sync_copy(idxf.at[pl.ds((b * 8 + 0) * N, N)], i0_v)
    pltpu.sync_copy(idxf.at[pl.ds((b * 8 + 1) * N, N)], i1_v)
    pltpu.sync_copy(idxf.at[pl.ds((b * 8 + 2) * N, N)], i2_v)
    pltpu.sync_copy(wf.at[pl.ds((b * 8 + 0) * N, N)], w0_v)
    pltpu.sync_copy(wf.at[pl.ds((b * 8 + 1) * N, N)], w1_v)
    pltpu.sync_copy(wf.at[pl.ds((b * 8 + 2) * N, N)], w2_v)

    def chunk(ci, _):
        base = ci * G
        c0 = pltpu.async_copy(g1f.at[i0_v.at[pl.ds(base, G)]], r0_v, sem)
        c1 = pltpu.async_copy(g1f.at[i1_v.at[pl.ds(base, G)]], r1_v, sem)
        c2 = pltpu.async_copy(g1f.at[i2_v.at[pl.ds(base, G)]], r2_v, sem)
        c0.wait()
        c1.wait()
        c2.wait()

        dn = lax.GatherDimensionNumbers(offset_dims=(), collapsed_slice_dims=(0,),
                                        start_index_map=(0,))
        splat = lambda vec, jidx: lax.gather(
            vec, jidx[:, None], dn, (1,),
            mode=lax.GatherScatterMode.PROMISE_IN_BOUNDS)

        def rowgrp(r, carry):
            g0 = r * 16
            wa0 = w0_v[pl.ds(base + g0, 16)]
            wa1 = w1_v[pl.ds(base + g0, 16)]
            wa2 = w2_v[pl.ds(base + g0, 16)]

            def row(j, carry2):
                g = g0 + j
                jidx = jnp.full((16,), j, jnp.int32)
                wv0 = splat(wa0, jidx)
                wv1 = splat(wa1, jidx)
                wv2 = splat(wa2, jidx)
                for c in range(ccols):
                    sl = pl.ds(c * 16, 16)
                    r0_v[g, sl] = (r0_v[g, sl] * wv0 + r1_v[g, sl] * wv1
                                   + r2_v[g, sl] * wv2)
                return carry2

            lax.fori_loop(0, 16, row, None, unroll=4)
            return carry

        lax.fori_loop(0, G // 16, rowgrp, None)
        pltpu.sync_copy(r0_v, up.at[pl.ds(b * N + base, G)])
        return _

    lax.fori_loop(0, nchunks, chunk, None, unroll=False)


def _bp_body(up, b1a, w1b, b1b, out):
    h = jnp.maximum(up[...] + b1a[...], 0.0)
    out[...] = jnp.dot(h, w1b[...], preferred_element_type=_F32) + b1b[...]


def kernel(feat1, feat2, feat3, ctr1, ctr2, ctr3, xyz,
           w3a, b3a, w3b, b3b, w2a, b2a, w2b, b2b, w1a, b1a, w1b, b1b):
    B, N, C = feat1.shape[0], xyz.shape[1], feat1.shape[2]
    S1, S2, S3 = feat1.shape[1], feat2.shape[1], feat3.shape[1]

    ctr1T = jnp.swapaxes(ctr1, 1, 2)
    ctr2T = jnp.swapaxes(ctr2, 1, 2)
    xyzT = jnp.swapaxes(xyz, 1, 2)
    w3at, w3ab = w3a[:C], w3a[C:]
    w2at, w2ab = w2a[:C], w2a[C:]
    b3a2 = b3a.reshape(1, C)
    b3b2 = b3b.reshape(1, C)
    b2a2 = b2a.reshape(1, C)
    b2b2 = b2b.reshape(1, C)
    b1a2 = b1a.reshape(1, C)
    b1b2 = b1b.reshape(1, C)

    full = lambda shape: pl.BlockSpec(shape, lambda *_: (0,) * len(shape))
    batch = lambda shape: pl.BlockSpec((1,) + shape,
                                       lambda b, *_: (b,) + (0,) * len(shape))

    g1 = pl.pallas_call(
        _a1_body,
        grid=(B,),
        in_specs=[
            batch((S1, C)), batch((S2, C)), batch((S3, C)),
            batch((S2, 3)), batch((S3, 3)),
            batch((3, S1)), batch((3, S2)),
            full((C, C)), full((C, C)), full((1, C)), full((C, C)), full((1, C)),
            full((C, C)), full((C, C)), full((1, C)), full((C, C)), full((1, C)),
            full((C, C)),
        ],
        out_specs=batch((S1, C)),
        out_shape=jax.ShapeDtypeStruct((B, S1, C), _F32),
    )(feat1, feat2, feat3, ctr2, ctr3, ctr1T, ctr2T,
      w3at, w3ab, b3a2, w3b, b3b2, w2at, w2ab, b2a2, w2b, b2b2, w1a)

    BLK_A = 1024
    idx8, w8 = pl.pallas_call(
        functools.partial(_a2_body, S=S1),
        grid=(B, N // BLK_A),
        in_specs=[
            pl.BlockSpec((1, S1, 3), lambda b, n: (b, 0, 0)),
            pl.BlockSpec((1, 3, BLK_A), lambda b, n: (b, 0, n)),
        ],
        out_specs=[
            pl.BlockSpec((1, 8, BLK_A), lambda b, n: (b, 0, n)),
            pl.BlockSpec((1, 8, BLK_A), lambda b, n: (b, 0, n)),
        ],
        out_shape=[
            jax.ShapeDtypeStruct((B, 8, N), jnp.int32),
            jax.ShapeDtypeStruct((B, 8, N), _F32),
        ],
    )(ctr1, xyzT)

    G = 64
    info = plsc.get_sparse_core_info()
    NC = info.num_cores
    g1f = g1.reshape(B * S1, C)
    idxf = idx8.reshape(B * 8 * N)
    wf = w8.reshape(B * 8 * N)
    mesh = plsc.VectorSubcoreMesh(core_axis_name="c", subcore_axis_name="s")
    up = pl.kernel(
        functools.partial(_sc_gather_body, N=N, C=C, G=G, NC=NC),
        mesh=mesh,
        out_type=jax.ShapeDtypeStruct((B * N, C), _F32),
        scratch_types=[
            pltpu.VMEM((N,), jnp.int32), pltpu.VMEM((N,), jnp.int32),
            pltpu.VMEM((N,), jnp.int32),
            pltpu.VMEM((N,), _F32), pltpu.VMEM((N,), _F32),
            pltpu.VMEM((N,), _F32),
            pltpu.VMEM((G, C), _F32), pltpu.VMEM((G, C), _F32),
            pltpu.VMEM((G, C), _F32),
            pltpu.SemaphoreType.DMA,
        ],
    )(g1f, idxf, wf)

    BLK_B = 1024
    out = pl.pallas_call(
        _bp_body,
        grid=(B * N // BLK_B,),
        in_specs=[
            pl.BlockSpec((BLK_B, C), lambda n: (n, 0)),
            pl.BlockSpec((1, C), lambda n: (0, 0)),
            pl.BlockSpec((C, C), lambda n: (0, 0)),
            pl.BlockSpec((1, C), lambda n: (0, 0)),
        ],
        out_specs=pl.BlockSpec((BLK_B, C), lambda n: (n, 0)),
        out_shape=jax.ShapeDtypeStruct((B * N, C), _F32),
    )(up, b1a2, w1b, b1b2)
    return out.reshape(B, N, C)


# speedup vs baseline: 1.4762x; 1.2563x over previous
import functools

import jax
import jax.numpy as jnp
from jax import lax
from jax.experimental import pallas as pl
from jax.experimental.pallas import tpu as pltpu
from jax.experimental.pallas import tpu_sc as plsc

_F32 = jnp.float32


def _top3_axis0(dist, S):
    iota = lax.broadcasted_iota(jnp.int32, dist.shape, 0)
    inf = jnp.array(jnp.inf, _F32)
    m1 = jnp.min(dist, axis=0, keepdims=True)
    i1 = jnp.min(jnp.where(dist == m1, iota, S), axis=0, keepdims=True)
    d1 = jnp.where(iota == i1, inf, dist)
    m2 = jnp.min(d1, axis=0, keepdims=True)
    i2 = jnp.min(jnp.where(d1 == m2, iota, S), axis=0, keepdims=True)
    d2 = jnp.where(iota == i2, inf, d1)
    m3 = jnp.min(d2, axis=0, keepdims=True)
    i3 = jnp.min(jnp.where(d2 == m3, iota, S), axis=0, keepdims=True)
    return (m1, m2, m3), (i1, i2, i3)


def _inv_dist_weights(m1, m2, m3):
    d1 = jnp.maximum(m1, 1e-8)
    d2 = jnp.maximum(m2, 1e-8)
    d3 = jnp.maximum(m3, 1e-8)
    w1 = 1.0 / d1
    w2 = 1.0 / d2
    w3 = 1.0 / d3
    s = w1 + w2 + w3
    return w1 / s, w2 / s, w3 / s


def _onehot_t(i123, w123, S, T):
    iota = lax.broadcasted_iota(jnp.int32, (S, T), 0)
    zero = jnp.array(0.0, _F32)
    (i1, i2, i3), (w1, w2, w3) = i123, w123
    return (jnp.where(iota == i1, w1, zero)
            + jnp.where(iota == i2, w2, zero)
            + jnp.where(iota == i3, w3, zero))


def _dist_matrix(src, dstT):
    cross = jnp.dot(src, dstT, preferred_element_type=_F32)
    s2 = jnp.sum(src * src, axis=1, keepdims=True)
    t2 = jnp.sum(dstT * dstT, axis=0, keepdims=True)
    d2 = s2 + t2 - 2.0 * cross
    return jnp.sqrt(jnp.maximum(d2, 0.0))


def _interp_t(ohT, g):
    return lax.dot_general(ohT, g, (((0,), (0,)), ((), ())),
                           preferred_element_type=_F32)


def _a1_body(f1, f2, f3, c2, c3, c1T, c2T,
             w3at, w3ab, b3a, w3b, b3b,
             w2at, w2ab, b2a, w2b, b2b, w1a, g1_out):
    f1v, f2v, f3v = f1[0], f2[0], f3[0]
    c2v, c3v = c2[0], c3[0]
    c1Tv, c2Tv = c1T[0], c2T[0]
    S3, S2, S1 = f3v.shape[0], f2v.shape[0], f1v.shape[0]

    dist = _dist_matrix(c3v, c2Tv)
    ms, is_ = _top3_axis0(dist, S3)
    ws = _inv_dist_weights(*ms)
    ohT = _onehot_t(is_, ws, S3, S2)
    g3 = jnp.dot(f3v, w3ab[...], preferred_element_type=_F32)
    up = _interp_t(ohT, g3)
    skip = jnp.dot(f2v, w3at[...], preferred_element_type=_F32)
    h = jnp.maximum(skip + up + b3a[...], 0.0)
    fused2 = jnp.dot(h, w3b[...], preferred_element_type=_F32) + b3b[...]

    dist = _dist_matrix(c2v, c1Tv)
    ms, is_ = _top3_axis0(dist, S2)
    ws = _inv_dist_weights(*ms)
    ohT = _onehot_t(is_, ws, S2, S1)
    g2 = jnp.dot(fused2, w2ab[...], preferred_element_type=_F32)
    up = _interp_t(ohT, g2)
    skip = jnp.dot(f1v, w2at[...], preferred_element_type=_F32)
    h = jnp.maximum(skip + up + b2a[...], 0.0)
    fused1 = jnp.dot(h, w2b[...], preferred_element_type=_F32) + b2b[...]

    g1_out[0] = jnp.dot(fused1, w1a[...], preferred_element_type=_F32)


def _a2_body(c1, xT, idx_out, w_out, *, S, OFFB):
    c1v = c1[0]
    xTv = xT[0]
    blk = xTv.shape[1]
    dist = _dist_matrix(c1v, xTv)
    (m1, m2, m3), (i1, i2, i3) = _top3_axis0(dist, S)
    w1, w2, w3 = _inv_dist_weights(m1, m2, m3)
    off = (pl.program_id(0) + OFFB) * S
    zi = jnp.zeros((5, blk), jnp.int32)
    zf = jnp.zeros((5, blk), _F32)
    idx_out[0] = jnp.concatenate([i1 + off, i2 + off, i3 + off, zi], axis=0)
    w_out[0] = jnp.concatenate([w1, w2, w3, zf], axis=0)


def _sc_gather_body(g1f, idxf, wf, up, i0_v, i1_v, i2_v, w0_v, w1_v, w2_v,
                    r0_v, r1_v, r2_v, sem, *, N, C, G, NC, L):
    w = lax.axis_index("s") * NC + lax.axis_index("c")
    b = (w * L) // N
    n0 = (w * L) % N
    nchunks = L // G
    ccols = C // 16

    pltpu.sync_copy(idxf.at[pl.ds((b * 8 + 0) * N + n0, L)], i0_v)
    pltpu.sync_copy(idxf.at[pl.ds((b * 8 + 1) * N + n0, L)], i1_v)
    pltpu.sync_copy(idxf.at[pl.ds((b * 8 + 2) * N + n0, L)], i2_v)
    pltpu.sync_copy(wf.at[pl.ds((b * 8 + 0) * N + n0, L)], w0_v)
    pltpu.sync_copy(wf.at[pl.ds((b * 8 + 1) * N + n0, L)], w1_v)
    pltpu.sync_copy(wf.at[pl.ds((b * 8 + 2) * N + n0, L)], w2_v)

    def chunk(ci, _):
        base = ci * G
        c0 = pltpu.async_copy(g1f.at[i0_v.at[pl.ds(base, G)]], r0_v, sem)
        c1 = pltpu.async_copy(g1f.at[i1_v.at[pl.ds(base, G)]], r1_v, sem)
        c2 = pltpu.async_copy(g1f.at[i2_v.at[pl.ds(base, G)]], r2_v, sem)
        c0.wait()
        c1.wait()
        c2.wait()

        dn = lax.GatherDimensionNumbers(offset_dims=(), collapsed_slice_dims=(0,),
                                        start_index_map=(0,))
        splat = lambda vec, jidx: lax.gather(
            vec, jidx[:, None], dn, (1,),
            mode=lax.GatherScatterMode.PROMISE_IN_BOUNDS)

        def rowgrp(r, carry):
            g0 = r * 16
            wa0 = w0_v[pl.ds(base + g0, 16)]
            wa1 = w1_v[pl.ds(base + g0, 16)]
            wa2 = w2_v[pl.ds(base + g0, 16)]

            def row(j, carry2):
                g = g0 + j
                jidx = jnp.full((16,), j, jnp.int32)
                wv0 = splat(wa0, jidx)
                wv1 = splat(wa1, jidx)
                wv2 = splat(wa2, jidx)
                for c in range(ccols):
                    sl = pl.ds(c * 16, 16)
                    r0_v[g, sl] = (r0_v[g, sl] * wv0 + r1_v[g, sl] * wv1
                                   + r2_v[g, sl] * wv2)
                return carry2

            lax.fori_loop(0, 16, row, None, unroll=4)
            return carry

        lax.fori_loop(0, G // 16, rowgrp, None)
        pltpu.sync_copy(r0_v, up.at[pl.ds(w * L + base, G)])
        return _

    lax.fori_loop(0, nchunks, chunk, None, unroll=False)


def _bp_body(up, b1a, w1b, b1b, out):
    h = jnp.maximum(up[...] + b1a[...], 0.0)
    out[...] = jnp.dot(h, w1b[...], preferred_element_type=_F32) + b1b[...]


def _bp_body2(up, b1a, w1b, b1b, prev, out):
    del prev
    h = jnp.maximum(up[...] + b1a[...], 0.0)
    out[...] = jnp.dot(h, w1b[...], preferred_element_type=_F32) + b1b[...]


def kernel(feat1, feat2, feat3, ctr1, ctr2, ctr3, xyz,
           w3a, b3a, w3b, b3b, w2a, b2a, w2b, b2b, w1a, b1a, w1b, b1b):
    B, N, C = feat1.shape[0], xyz.shape[1], feat1.shape[2]
    S1, S2, S3 = feat1.shape[1], feat2.shape[1], feat3.shape[1]

    ctr1T = jnp.swapaxes(ctr1, 1, 2)
    ctr2T = jnp.swapaxes(ctr2, 1, 2)
    xyzT = jnp.swapaxes(xyz, 1, 2)
    w3at, w3ab = w3a[:C], w3a[C:]
    w2at, w2ab = w2a[:C], w2a[C:]
    b3a2 = b3a.reshape(1, C)
    b3b2 = b3b.reshape(1, C)
    b2a2 = b2a.reshape(1, C)
    b2b2 = b2b.reshape(1, C)
    b1a2 = b1a.reshape(1, C)
    b1b2 = b1b.reshape(1, C)

    full = lambda shape: pl.BlockSpec(shape, lambda *_: (0,) * len(shape))
    batch = lambda shape: pl.BlockSpec((1,) + shape,
                                       lambda b, *_: (b,) + (0,) * len(shape))

    g1 = pl.pallas_call(
        _a1_body,
        grid=(B,),
        in_specs=[
            batch((S1, C)), batch((S2, C)), batch((S3, C)),
            batch((S2, 3)), batch((S3, 3)),
            batch((3, S1)), batch((3, S2)),
            full((C, C)), full((C, C)), full((1, C)), full((C, C)), full((1, C)),
            full((C, C)), full((C, C)), full((1, C)), full((C, C)), full((1, C)),
            full((C, C)),
        ],
        out_specs=batch((S1, C)),
        out_shape=jax.ShapeDtypeStruct((B, S1, C), _F32),
    )(feat1, feat2, feat3, ctr2, ctr3, ctr1T, ctr2T,
      w3at, w3ab, b3a2, w3b, b3b2, w2at, w2ab, b2a2, w2b, b2b2, w1a)

    BLK_A = 1024
    BLK_B = 1024
    G = 64
    info = plsc.get_sparse_core_info()
    NC = info.num_cores
    NW = NC * info.num_subcores
    BH = B // 2
    L = BH * N // NW
    g1f = g1.reshape(B * S1, C)
    mesh = plsc.VectorSubcoreMesh(core_axis_name="c", subcore_axis_name="s")

    ups = []
    for h in range(2):
        sl = slice(h * BH, (h + 1) * BH)
        idx8, w8 = pl.pallas_call(
            functools.partial(_a2_body, S=S1, OFFB=h * BH),
            grid=(BH, N // BLK_A),
            in_specs=[
                pl.BlockSpec((1, S1, 3), lambda b, n: (b, 0, 0)),
                pl.BlockSpec((1, 3, BLK_A), lambda b, n: (b, 0, n)),
            ],
            out_specs=[
                pl.BlockSpec((1, 8, BLK_A), lambda b, n: (b, 0, n)),
                pl.BlockSpec((1, 8, BLK_A), lambda b, n: (b, 0, n)),
            ],
            out_shape=[
                jax.ShapeDtypeStruct((BH, 8, N), jnp.int32),
                jax.ShapeDtypeStruct((BH, 8, N), _F32),
            ],
        )(ctr1[sl], xyzT[sl])

        up = pl.kernel(
            functools.partial(_sc_gather_body, N=N, C=C, G=G, NC=NC, L=L),
            mesh=mesh,
            out_type=jax.ShapeDtypeStruct((BH * N, C), _F32),
            scratch_types=[
                pltpu.VMEM((L,), jnp.int32), pltpu.VMEM((L,), jnp.int32),
                pltpu.VMEM((L,), jnp.int32),
                pltpu.VMEM((L,), _F32), pltpu.VMEM((L,), _F32),
                pltpu.VMEM((L,), _F32),
                pltpu.VMEM((G, C), _F32), pltpu.VMEM((G, C), _F32),
                pltpu.VMEM((G, C), _F32),
                pltpu.SemaphoreType.DMA,
            ],
        )(g1f, idx8.reshape(BH * 8 * N), w8.reshape(BH * 8 * N))
        ups.append(up)

    PH = BH * N // BLK_B
    out = pl.pallas_call(
        _bp_body,
        grid=(PH,),
        in_specs=[
            pl.BlockSpec((BLK_B, C), lambda n: (n, 0)),
            pl.BlockSpec((1, C), lambda n: (0, 0)),
            pl.BlockSpec((C, C), lambda n: (0, 0)),
            pl.BlockSpec((1, C), lambda n: (0, 0)),
        ],
        out_specs=pl.BlockSpec((BLK_B, C), lambda n: (n, 0)),
        out_shape=jax.ShapeDtypeStruct((B * N, C), _F32),
    )(ups[0], b1a2, w1b, b1b2)
    out = pl.pallas_call(
        _bp_body2,
        grid=(PH,),
        in_specs=[
            pl.BlockSpec((BLK_B, C), lambda n: (n, 0)),
            pl.BlockSpec((1, C), lambda n: (0, 0)),
            pl.BlockSpec((C, C), lambda n: (0, 0)),
            pl.BlockSpec((1, C), lambda n: (0, 0)),
            pl.BlockSpec((BLK_B, C), lambda n: (0, 0)),
        ],
        out_specs=pl.BlockSpec((BLK_B, C), lambda n: (n + PH, 0)),
        out_shape=jax.ShapeDtypeStruct((B * N, C), _F32),
        input_output_aliases={4: 0},
    )(ups[1], b1a2, w1b, b1b2, out)
    return out.reshape(B, N, C)


# speedup vs baseline: 1.6599x; 1.1244x over previous
import functools

import jax
import jax.numpy as jnp
from jax import lax
from jax.experimental import pallas as pl
from jax.experimental.pallas import tpu as pltpu
from jax.experimental.pallas import tpu_sc as plsc

_F32 = jnp.float32


def _top3_axis0(dist, S):
    iota = lax.broadcasted_iota(jnp.int32, dist.shape, 0)
    inf = jnp.array(jnp.inf, _F32)
    m1 = jnp.min(dist, axis=0, keepdims=True)
    i1 = jnp.min(jnp.where(dist == m1, iota, S), axis=0, keepdims=True)
    d1 = jnp.where(iota == i1, inf, dist)
    m2 = jnp.min(d1, axis=0, keepdims=True)
    i2 = jnp.min(jnp.where(d1 == m2, iota, S), axis=0, keepdims=True)
    d2 = jnp.where(iota == i2, inf, d1)
    m3 = jnp.min(d2, axis=0, keepdims=True)
    i3 = jnp.min(jnp.where(d2 == m3, iota, S), axis=0, keepdims=True)
    return (m1, m2, m3), (i1, i2, i3)


def _inv_dist_weights(m1, m2, m3):
    d1 = jnp.maximum(m1, 1e-8)
    d2 = jnp.maximum(m2, 1e-8)
    d3 = jnp.maximum(m3, 1e-8)
    w1 = 1.0 / d1
    w2 = 1.0 / d2
    w3 = 1.0 / d3
    s = w1 + w2 + w3
    return w1 / s, w2 / s, w3 / s


def _onehot_t(i123, w123, S, T):
    iota = lax.broadcasted_iota(jnp.int32, (S, T), 0)
    zero = jnp.array(0.0, _F32)
    (i1, i2, i3), (w1, w2, w3) = i123, w123
    return (jnp.where(iota == i1, w1, zero)
            + jnp.where(iota == i2, w2, zero)
            + jnp.where(iota == i3, w3, zero))


def _dist_matrix(src, dstT):
    cross = jnp.dot(src, dstT, preferred_element_type=_F32)
    s2 = jnp.sum(src * src, axis=1, keepdims=True)
    t2 = jnp.sum(dstT * dstT, axis=0, keepdims=True)
    d2 = s2 + t2 - 2.0 * cross
    return jnp.sqrt(jnp.maximum(d2, 0.0))


def _interp_t(ohT, g):
    return lax.dot_general(ohT, g, (((0,), (0,)), ((), ())),
                           preferred_element_type=_F32)


def _a1_body(f1, f2, f3, c2, c3, c1T, c2T,
             w3at, w3ab, b3a, w3b, b3b,
             w2at, w2ab, b2a, w2b, b2b, w1a, g1_out):
    f1v, f2v, f3v = f1[0], f2[0], f3[0]
    c2v, c3v = c2[0], c3[0]
    c1Tv, c2Tv = c1T[0], c2T[0]
    S3, S2, S1 = f3v.shape[0], f2v.shape[0], f1v.shape[0]

    dist = _dist_matrix(c3v, c2Tv)
    ms, is_ = _top3_axis0(dist, S3)
    ws = _inv_dist_weights(*ms)
    ohT = _onehot_t(is_, ws, S3, S2)
    g3 = jnp.dot(f3v, w3ab[...], preferred_element_type=_F32)
    up = _interp_t(ohT, g3)
    skip = jnp.dot(f2v, w3at[...], preferred_element_type=_F32)
    h = jnp.maximum(skip + up + b3a[...], 0.0)
    fused2 = jnp.dot(h, w3b[...], preferred_element_type=_F32) + b3b[...]

    dist = _dist_matrix(c2v, c1Tv)
    ms, is_ = _top3_axis0(dist, S2)
    ws = _inv_dist_weights(*ms)
    ohT = _onehot_t(is_, ws, S2, S1)
    g2 = jnp.dot(fused2, w2ab[...], preferred_element_type=_F32)
    up = _interp_t(ohT, g2)
    skip = jnp.dot(f1v, w2at[...], preferred_element_type=_F32)
    h = jnp.maximum(skip + up + b2a[...], 0.0)
    fused1 = jnp.dot(h, w2b[...], preferred_element_type=_F32) + b2b[...]

    g1_out[0] = jnp.dot(fused1, w1a[...], preferred_element_type=_F32)


def _a2_body(c1, xT, idx_out, w_out, *, S, OFFB):
    c1v = c1[0]
    xTv = xT[0]
    blk = xTv.shape[1]
    dist = _dist_matrix(c1v, xTv)
    (m1, m2, m3), (i1, i2, i3) = _top3_axis0(dist, S)
    w1, w2, w3 = _inv_dist_weights(m1, m2, m3)
    off = (pl.program_id(0) + OFFB) * S
    zi = jnp.zeros((5, blk), jnp.int32)
    zf = jnp.zeros((5, blk), _F32)
    idx_out[0] = jnp.concatenate([i1 + off, i2 + off, i3 + off, zi], axis=0)
    w_out[0] = jnp.concatenate([w1, w2, w3, zf], axis=0)


def _sc_gather_body(g1f, idxf, wf, up, i0_v, i1_v, i2_v, w0_v, w1_v, w2_v,
                    r0_v, r1_v, r2_v, sem, *, N, C, G, NC, L):
    w = lax.axis_index("s") * NC + lax.axis_index("c")
    b = (w * L) // N
    n0 = (w * L) % N
    nchunks = L // G
    ccols = C // 16

    pltpu.sync_copy(idxf.at[pl.ds((b * 8 + 0) * N + n0, L)], i0_v)
    pltpu.sync_copy(idxf.at[pl.ds((b * 8 + 1) * N + n0, L)], i1_v)
    pltpu.sync_copy(idxf.at[pl.ds((b * 8 + 2) * N + n0, L)], i2_v)
    pltpu.sync_copy(wf.at[pl.ds((b * 8 + 0) * N + n0, L)], w0_v)
    pltpu.sync_copy(wf.at[pl.ds((b * 8 + 1) * N + n0, L)], w1_v)
    pltpu.sync_copy(wf.at[pl.ds((b * 8 + 2) * N + n0, L)], w2_v)

    def chunk(ci, _):
        base = ci * G
        c0 = pltpu.async_copy(g1f.at[i0_v.at[pl.ds(base, G)]], r0_v, sem)
        c1 = pltpu.async_copy(g1f.at[i1_v.at[pl.ds(base, G)]], r1_v, sem)
        c2 = pltpu.async_copy(g1f.at[i2_v.at[pl.ds(base, G)]], r2_v, sem)
        c0.wait()
        c1.wait()
        c2.wait()

        dn = lax.GatherDimensionNumbers(offset_dims=(), collapsed_slice_dims=(0,),
                                        start_index_map=(0,))
        splat = lambda vec, jidx: lax.gather(
            vec, jidx[:, None], dn, (1,),
            mode=lax.GatherScatterMode.PROMISE_IN_BOUNDS)

        def rowgrp(r, carry):
            g0 = r * 16
            wa0 = w0_v[pl.ds(base + g0, 16)]
            wa1 = w1_v[pl.ds(base + g0, 16)]
            wa2 = w2_v[pl.ds(base + g0, 16)]

            def row(j, carry2):
                g = g0 + j
                jidx = jnp.full((16,), j, jnp.int32)
                wv0 = splat(wa0, jidx)
                wv1 = splat(wa1, jidx)
                wv2 = splat(wa2, jidx)
                for c in range(ccols):
                    sl = pl.ds(c * 16, 16)
                    r0_v[g, sl] = (r0_v[g, sl] * wv0 + r1_v[g, sl] * wv1
                                   + r2_v[g, sl] * wv2)
                return carry2

            lax.fori_loop(0, 16, row, None, unroll=4)
            return carry

        lax.fori_loop(0, G // 16, rowgrp, None)
        pltpu.sync_copy(r0_v, up.at[pl.ds(w * L + base, G)])
        return _

    lax.fori_loop(0, nchunks, chunk, None, unroll=False)


def _bp_body(up, b1a, w1b, b1b, out):
    h = jnp.maximum(up[...] + b1a[...], 0.0)
    out[...] = jnp.dot(h, w1b[...], preferred_element_type=_F32) + b1b[...]


def _bp_body2(up, b1a, w1b, b1b, prev, out):
    del prev
    h = jnp.maximum(up[...] + b1a[...], 0.0)
    out[...] = jnp.dot(h, w1b[...], preferred_element_type=_F32) + b1b[...]


def kernel(feat1, feat2, feat3, ctr1, ctr2, ctr3, xyz,
           w3a, b3a, w3b, b3b, w2a, b2a, w2b, b2b, w1a, b1a, w1b, b1b):
    B, N, C = feat1.shape[0], xyz.shape[1], feat1.shape[2]
    S1, S2, S3 = feat1.shape[1], feat2.shape[1], feat3.shape[1]

    ctr1T = jnp.swapaxes(ctr1, 1, 2)
    ctr2T = jnp.swapaxes(ctr2, 1, 2)
    xyzT = jnp.swapaxes(xyz, 1, 2)
    w3at, w3ab = w3a[:C], w3a[C:]
    w2at, w2ab = w2a[:C], w2a[C:]
    b3a2 = b3a.reshape(1, C)
    b3b2 = b3b.reshape(1, C)
    b2a2 = b2a.reshape(1, C)
    b2b2 = b2b.reshape(1, C)
    b1a2 = b1a.reshape(1, C)
    b1b2 = b1b.reshape(1, C)

    full = lambda shape: pl.BlockSpec(shape, lambda *_: (0,) * len(shape))
    batch = lambda shape: pl.BlockSpec((1,) + shape,
                                       lambda b, *_: (b,) + (0,) * len(shape))

    g1 = pl.pallas_call(
        _a1_body,
        grid=(B,),
        in_specs=[
            batch((S1, C)), batch((S2, C)), batch((S3, C)),
            batch((S2, 3)), batch((S3, 3)),
            batch((3, S1)), batch((3, S2)),
            full((C, C)), full((C, C)), full((1, C)), full((C, C)), full((1, C)),
            full((C, C)), full((C, C)), full((1, C)), full((C, C)), full((1, C)),
            full((C, C)),
        ],
        out_specs=batch((S1, C)),
        out_shape=jax.ShapeDtypeStruct((B, S1, C), _F32),
    )(feat1, feat2, feat3, ctr2, ctr3, ctr1T, ctr2T,
      w3at, w3ab, b3a2, w3b, b3b2, w2at, w2ab, b2a2, w2b, b2b2, w1a)

    BLK_A = 1024
    BLK_B = 1024
    G = 64
    NSPLIT = 4
    info = plsc.get_sparse_core_info()
    NC = info.num_cores
    NW = NC * info.num_subcores
    BH = B // NSPLIT
    L = BH * N // NW
    g1f = g1.reshape(B * S1, C)
    mesh = plsc.VectorSubcoreMesh(core_axis_name="c", subcore_axis_name="s")

    ups = []
    for h in range(NSPLIT):
        sl = slice(h * BH, (h + 1) * BH)
        idx8, w8 = pl.pallas_call(
            functools.partial(_a2_body, S=S1, OFFB=h * BH),
            grid=(BH, N // BLK_A),
            in_specs=[
                pl.BlockSpec((1, S1, 3), lambda b, n: (b, 0, 0)),
                pl.BlockSpec((1, 3, BLK_A), lambda b, n: (b, 0, n)),
            ],
            out_specs=[
                pl.BlockSpec((1, 8, BLK_A), lambda b, n: (b, 0, n)),
                pl.BlockSpec((1, 8, BLK_A), lambda b, n: (b, 0, n)),
            ],
            out_shape=[
                jax.ShapeDtypeStruct((BH, 8, N), jnp.int32),
                jax.ShapeDtypeStruct((BH, 8, N), _F32),
            ],
        )(ctr1[sl], xyzT[sl])

        up = pl.kernel(
            functools.partial(_sc_gather_body, N=N, C=C, G=G, NC=NC, L=L),
            mesh=mesh,
            out_type=jax.ShapeDtypeStruct((BH * N, C), _F32),
            scratch_types=[
                pltpu.VMEM((L,), jnp.int32), pltpu.VMEM((L,), jnp.int32),
                pltpu.VMEM((L,), jnp.int32),
                pltpu.VMEM((L,), _F32), pltpu.VMEM((L,), _F32),
                pltpu.VMEM((L,), _F32),
                pltpu.VMEM((G, C), _F32), pltpu.VMEM((G, C), _F32),
                pltpu.VMEM((G, C), _F32),
                pltpu.SemaphoreType.DMA,
            ],
        )(g1f, idx8.reshape(BH * 8 * N), w8.reshape(BH * 8 * N))
        ups.append(up)

    PH = BH * N // BLK_B
    out = pl.pallas_call(
        _bp_body,
        grid=(PH,),
        in_specs=[
            pl.BlockSpec((BLK_B, C), lambda n: (n, 0)),
            pl.BlockSpec((1, C), lambda n: (0, 0)),
            pl.BlockSpec((C, C), lambda n: (0, 0)),
            pl.BlockSpec((1, C), lambda n: (0, 0)),
        ],
        out_specs=pl.BlockSpec((BLK_B, C), lambda n: (n, 0)),
        out_shape=jax.ShapeDtypeStruct((B * N, C), _F32),
    )(ups[0], b1a2, w1b, b1b2)
    for h in range(1, NSPLIT):
        out = pl.pallas_call(
            _bp_body2,
            grid=(PH,),
            in_specs=[
                pl.BlockSpec((BLK_B, C), lambda n: (n, 0)),
                pl.BlockSpec((1, C), lambda n: (0, 0)),
                pl.BlockSpec((C, C), lambda n: (0, 0)),
                pl.BlockSpec((1, C), lambda n: (0, 0)),
                pl.BlockSpec((BLK_B, C), lambda n: (0, 0)),
            ],
            out_specs=pl.BlockSpec(
                (BLK_B, C), lambda n, _h=h: (n + _h * PH, 0)),
            out_shape=jax.ShapeDtypeStruct((B * N, C), _F32),
            input_output_aliases={4: 0},
        )(ups[h], b1a2, w1b, b1b2, out)
    return out.reshape(B, N, C)


# speedup vs baseline: 1.6828x; 1.0138x over previous
import functools

import jax
import jax.numpy as jnp
from jax import lax
from jax.experimental import pallas as pl
from jax.experimental.pallas import tpu as pltpu
from jax.experimental.pallas import tpu_sc as plsc

_F32 = jnp.float32


def _top3_axis0(dist, S):
    iota = lax.broadcasted_iota(jnp.int32, dist.shape, 0)
    inf = jnp.array(jnp.inf, _F32)
    m1 = jnp.min(dist, axis=0, keepdims=True)
    i1 = jnp.min(jnp.where(dist == m1, iota, S), axis=0, keepdims=True)
    d1 = jnp.where(iota == i1, inf, dist)
    m2 = jnp.min(d1, axis=0, keepdims=True)
    i2 = jnp.min(jnp.where(d1 == m2, iota, S), axis=0, keepdims=True)
    d2 = jnp.where(iota == i2, inf, d1)
    m3 = jnp.min(d2, axis=0, keepdims=True)
    i3 = jnp.min(jnp.where(d2 == m3, iota, S), axis=0, keepdims=True)
    return (m1, m2, m3), (i1, i2, i3)


def _inv_dist_weights(m1, m2, m3):
    d1 = jnp.maximum(m1, 1e-8)
    d2 = jnp.maximum(m2, 1e-8)
    d3 = jnp.maximum(m3, 1e-8)
    w1 = 1.0 / d1
    w2 = 1.0 / d2
    w3 = 1.0 / d3
    s = w1 + w2 + w3
    return w1 / s, w2 / s, w3 / s


def _onehot_t(i123, w123, S, T):
    iota = lax.broadcasted_iota(jnp.int32, (S, T), 0)
    zero = jnp.array(0.0, _F32)
    (i1, i2, i3), (w1, w2, w3) = i123, w123
    return (jnp.where(iota == i1, w1, zero)
            + jnp.where(iota == i2, w2, zero)
            + jnp.where(iota == i3, w3, zero))


def _dist_matrix(src, dstT):
    cross = jnp.dot(src, dstT, preferred_element_type=_F32)
    s2 = jnp.sum(src * src, axis=1, keepdims=True)
    t2 = jnp.sum(dstT * dstT, axis=0, keepdims=True)
    d2 = s2 + t2 - 2.0 * cross
    return jnp.sqrt(jnp.maximum(d2, 0.0))


def _interp_t(ohT, g):
    return lax.dot_general(ohT, g, (((0,), (0,)), ((), ())),
                           preferred_element_type=_F32)


def _a1_body(f1, f2, f3, c2, c3, c1T, c2T,
             w3at, w3ab, b3a, w3b, b3b,
             w2at, w2ab, b2a, w2b, b2b, w1a, g1_out):
    f1v, f2v, f3v = f1[0], f2[0], f3[0]
    c2v, c3v = c2[0], c3[0]
    c1Tv, c2Tv = c1T[0], c2T[0]
    S3, S2, S1 = f3v.shape[0], f2v.shape[0], f1v.shape[0]

    dist = _dist_matrix(c3v, c2Tv)
    ms, is_ = _top3_axis0(dist, S3)
    ws = _inv_dist_weights(*ms)
    ohT = _onehot_t(is_, ws, S3, S2)
    g3 = jnp.dot(f3v, w3ab[...], preferred_element_type=_F32)
    up = _interp_t(ohT, g3)
    skip = jnp.dot(f2v, w3at[...], preferred_element_type=_F32)
    h = jnp.maximum(skip + up + b3a[...], 0.0)
    fused2 = jnp.dot(h, w3b[...], preferred_element_type=_F32) + b3b[...]

    dist = _dist_matrix(c2v, c1Tv)
    ms, is_ = _top3_axis0(dist, S2)
    ws = _inv_dist_weights(*ms)
    ohT = _onehot_t(is_, ws, S2, S1)
    g2 = jnp.dot(fused2, w2ab[...], preferred_element_type=_F32)
    up = _interp_t(ohT, g2)
    skip = jnp.dot(f1v, w2at[...], preferred_element_type=_F32)
    h = jnp.maximum(skip + up + b2a[...], 0.0)
    fused1 = jnp.dot(h, w2b[...], preferred_element_type=_F32) + b2b[...]

    g1_out[0] = jnp.dot(fused1, w1a[...], preferred_element_type=_F32)


def _a2_body(c1, xT, idx_out, w_out, *, S, OFFB):
    c1v = c1[0]
    xTv = xT[0]
    blk = xTv.shape[1]
    dist = _dist_matrix(c1v, xTv)
    (m1, m2, m3), (i1, i2, i3) = _top3_axis0(dist, S)
    w1, w2, w3 = _inv_dist_weights(m1, m2, m3)
    off = (pl.program_id(0) + OFFB) * S
    zi = jnp.zeros((5, blk), jnp.int32)
    zf = jnp.zeros((5, blk), _F32)
    idx_out[0] = jnp.concatenate([i1 + off, i2 + off, i3 + off, zi], axis=0)
    w_out[0] = jnp.concatenate([w1, w2, w3, zf], axis=0)


def _sc_gather_body(g1f, idxf, wf, up, i0_v, i1_v, i2_v, w0_v, w1_v, w2_v,
                    r0_v, r1_v, r2_v, sem, *, N, C, G, NC, L):
    w = lax.axis_index("s") * NC + lax.axis_index("c")
    b = (w * L) // N
    n0 = (w * L) % N
    nchunks = L // G
    ccols = C // 16

    pltpu.sync_copy(idxf.at[pl.ds((b * 8 + 0) * N + n0, L)], i0_v)
    pltpu.sync_copy(idxf.at[pl.ds((b * 8 + 1) * N + n0, L)], i1_v)
    pltpu.sync_copy(idxf.at[pl.ds((b * 8 + 2) * N + n0, L)], i2_v)
    pltpu.sync_copy(wf.at[pl.ds((b * 8 + 0) * N + n0, L)], w0_v)
    pltpu.sync_copy(wf.at[pl.ds((b * 8 + 1) * N + n0, L)], w1_v)
    pltpu.sync_copy(wf.at[pl.ds((b * 8 + 2) * N + n0, L)], w2_v)

    def chunk(ci, _):
        base = ci * G
        c0 = pltpu.async_copy(g1f.at[i0_v.at[pl.ds(base, G)]], r0_v, sem)
        c1 = pltpu.async_copy(g1f.at[i1_v.at[pl.ds(base, G)]], r1_v, sem)
        c2 = pltpu.async_copy(g1f.at[i2_v.at[pl.ds(base, G)]], r2_v, sem)
        c0.wait()
        c1.wait()
        c2.wait()

        dn = lax.GatherDimensionNumbers(offset_dims=(), collapsed_slice_dims=(0,),
                                        start_index_map=(0,))
        splat = lambda vec, jidx: lax.gather(
            vec, jidx[:, None], dn, (1,),
            mode=lax.GatherScatterMode.PROMISE_IN_BOUNDS)

        def rowgrp(r, carry):
            g0 = r * 16
            wa0 = w0_v[pl.ds(base + g0, 16)]
            wa1 = w1_v[pl.ds(base + g0, 16)]
            wa2 = w2_v[pl.ds(base + g0, 16)]

            def row(j, carry2):
                g = g0 + j
                jidx = jnp.full((16,), j, jnp.int32)
                wv0 = splat(wa0, jidx)
                wv1 = splat(wa1, jidx)
                wv2 = splat(wa2, jidx)
                for c in range(ccols):
                    sl = pl.ds(c * 16, 16)
                    r0_v[g, sl] = (r0_v[g, sl] * wv0 + r1_v[g, sl] * wv1
                                   + r2_v[g, sl] * wv2)
                return carry2

            lax.fori_loop(0, 16, row, None, unroll=4)
            return carry

        lax.fori_loop(0, G // 16, rowgrp, None)
        pltpu.sync_copy(r0_v, up.at[pl.ds(w * L + base, G)])
        return _

    lax.fori_loop(0, nchunks, chunk, None, unroll=False)


def _bp_body(up, b1a, w1b, b1b, out):
    h = jnp.maximum(up[...] + b1a[...], 0.0)
    out[...] = jnp.dot(h, w1b[...], preferred_element_type=_F32) + b1b[...]


def _bp_body2(up, b1a, w1b, b1b, prev, out):
    del prev
    h = jnp.maximum(up[...] + b1a[...], 0.0)
    out[...] = jnp.dot(h, w1b[...], preferred_element_type=_F32) + b1b[...]


def kernel(feat1, feat2, feat3, ctr1, ctr2, ctr3, xyz,
           w3a, b3a, w3b, b3b, w2a, b2a, w2b, b2b, w1a, b1a, w1b, b1b):
    B, N, C = feat1.shape[0], xyz.shape[1], feat1.shape[2]
    S1, S2, S3 = feat1.shape[1], feat2.shape[1], feat3.shape[1]

    ctr1T = jnp.swapaxes(ctr1, 1, 2)
    ctr2T = jnp.swapaxes(ctr2, 1, 2)
    xyzT = jnp.swapaxes(xyz, 1, 2)
    w3at, w3ab = w3a[:C], w3a[C:]
    w2at, w2ab = w2a[:C], w2a[C:]
    b3a2 = b3a.reshape(1, C)
    b3b2 = b3b.reshape(1, C)
    b2a2 = b2a.reshape(1, C)
    b2b2 = b2b.reshape(1, C)
    b1a2 = b1a.reshape(1, C)
    b1b2 = b1b.reshape(1, C)

    full = lambda shape: pl.BlockSpec(shape, lambda *_: (0,) * len(shape))
    batch = lambda shape: pl.BlockSpec((1,) + shape,
                                       lambda b, *_: (b,) + (0,) * len(shape))

    g1 = pl.pallas_call(
        _a1_body,
        grid=(B,),
        in_specs=[
            batch((S1, C)), batch((S2, C)), batch((S3, C)),
            batch((S2, 3)), batch((S3, 3)),
            batch((3, S1)), batch((3, S2)),
            full((C, C)), full((C, C)), full((1, C)), full((C, C)), full((1, C)),
            full((C, C)), full((C, C)), full((1, C)), full((C, C)), full((1, C)),
            full((C, C)),
        ],
        out_specs=batch((S1, C)),
        out_shape=jax.ShapeDtypeStruct((B, S1, C), _F32),
    )(feat1, feat2, feat3, ctr2, ctr3, ctr1T, ctr2T,
      w3at, w3ab, b3a2, w3b, b3b2, w2at, w2ab, b2a2, w2b, b2b2, w1a)

    BLK_A = 1024
    BLK_B = 1024
    G = 64
    NSPLIT = 8
    info = plsc.get_sparse_core_info()
    NC = info.num_cores
    NW = NC * info.num_subcores
    BH = B // NSPLIT
    L = BH * N // NW
    g1f = g1.reshape(B * S1, C)
    mesh = plsc.VectorSubcoreMesh(core_axis_name="c", subcore_axis_name="s")

    ups = []
    for h in range(NSPLIT):
        sl = slice(h * BH, (h + 1) * BH)
        idx8, w8 = pl.pallas_call(
            functools.partial(_a2_body, S=S1, OFFB=h * BH),
            grid=(BH, N // BLK_A),
            in_specs=[
                pl.BlockSpec((1, S1, 3), lambda b, n: (b, 0, 0)),
                pl.BlockSpec((1, 3, BLK_A), lambda b, n: (b, 0, n)),
            ],
            out_specs=[
                pl.BlockSpec((1, 8, BLK_A), lambda b, n: (b, 0, n)),
                pl.BlockSpec((1, 8, BLK_A), lambda b, n: (b, 0, n)),
            ],
            out_shape=[
                jax.ShapeDtypeStruct((BH, 8, N), jnp.int32),
                jax.ShapeDtypeStruct((BH, 8, N), _F32),
            ],
        )(ctr1[sl], xyzT[sl])

        up = pl.kernel(
            functools.partial(_sc_gather_body, N=N, C=C, G=G, NC=NC, L=L),
            mesh=mesh,
            out_type=jax.ShapeDtypeStruct((BH * N, C), _F32),
            scratch_types=[
                pltpu.VMEM((L,), jnp.int32), pltpu.VMEM((L,), jnp.int32),
                pltpu.VMEM((L,), jnp.int32),
                pltpu.VMEM((L,), _F32), pltpu.VMEM((L,), _F32),
                pltpu.VMEM((L,), _F32),
                pltpu.VMEM((G, C), _F32), pltpu.VMEM((G, C), _F32),
                pltpu.VMEM((G, C), _F32),
                pltpu.SemaphoreType.DMA,
            ],
        )(g1f, idx8.reshape(BH * 8 * N), w8.reshape(BH * 8 * N))
        ups.append(up)

    PH = BH * N // BLK_B
    out = pl.pallas_call(
        _bp_body,
        grid=(PH,),
        in_specs=[
            pl.BlockSpec((BLK_B, C), lambda n: (n, 0)),
            pl.BlockSpec((1, C), lambda n: (0, 0)),
            pl.BlockSpec((C, C), lambda n: (0, 0)),
            pl.BlockSpec((1, C), lambda n: (0, 0)),
        ],
        out_specs=pl.BlockSpec((BLK_B, C), lambda n: (n, 0)),
        out_shape=jax.ShapeDtypeStruct((B * N, C), _F32),
    )(ups[0], b1a2, w1b, b1b2)
    for h in range(1, NSPLIT):
        out = pl.pallas_call(
            _bp_body2,
            grid=(PH,),
            in_specs=[
                pl.BlockSpec((BLK_B, C), lambda n: (n, 0)),
                pl.BlockSpec((1, C), lambda n: (0, 0)),
                pl.BlockSpec((C, C), lambda n: (0, 0)),
                pl.BlockSpec((1, C), lambda n: (0, 0)),
                pl.BlockSpec((BLK_B, C), lambda n: (0, 0)),
            ],
            out_specs=pl.BlockSpec(
                (BLK_B, C), lambda n, _h=h: (n + _h * PH, 0)),
            out_shape=jax.ShapeDtypeStruct((B * N, C), _F32),
            input_output_aliases={4: 0},
        )(ups[h], b1a2, w1b, b1b2, out)
    return out.reshape(B, N, C)
